# Initial kernel scaffold; baseline (speedup 1.0000x reference)
import jax
import jax.numpy as jnp
from jax.experimental import pallas as pl


def kernel(x, event, W1, b1, W2, b2, W3, b3, W4, b4):
    raise NotImplementedError("write your pallas kernel here")


# speedup vs baseline: 4.9430x; 4.9430x over previous
import functools

import jax
import jax.numpy as jnp
from jax.experimental import pallas as pl
from jax.experimental.pallas import tpu as pltpu

N = 100000
D = 128
G = 32
NEV = 512
P = 1000
NB = N // P


def _body(ev_ref, x_ref, w1_ref, b1_ref, w2_ref, b2_ref, w3_ref, b3_ref,
          w4_ref, b4_ref, out_ref, acc_ref, cnt_ref):
    i = pl.program_id(0)

    @pl.when(i == 0)
    def _init():
        acc_ref[...] = jnp.zeros_like(acc_ref)
        cnt_ref[...] = jnp.zeros_like(cnt_ref)

    x = x_ref[...]
    h = jnp.maximum(
        jnp.dot(x, w1_ref[...], preferred_element_type=jnp.float32)
        + b1_ref[...], 0.0)
    h = (jnp.dot(h, w2_ref[...], preferred_element_type=jnp.float32)
         + b2_ref[...])

    ev = ev_ref[0]
    evb = jnp.broadcast_to(ev, (NEV, P))
    seg = jax.lax.broadcasted_iota(jnp.int32, (NEV, P), 0)
    et = (evb == seg).astype(jnp.float32)
    acc_ref[...] += jnp.dot(et, h, preferred_element_type=jnp.float32)
    cnt_ref[...] += jnp.sum(et, axis=1, keepdims=True)

    @pl.when(i == NB - 1)
    def _finish():
        gsum = acc_ref[...]
        cnt = jnp.maximum(cnt_ref[...], 1.0)
        g = jnp.concatenate([gsum, gsum / cnt], axis=1)
        t = jnp.maximum(
            jnp.dot(g, w3_ref[...], preferred_element_type=jnp.float32)
            + b3_ref[...], 0.0)
        out_ref[...] = (jnp.dot(t, w4_ref[...],
                                preferred_element_type=jnp.float32)
                        + b4_ref[...])


@functools.partial(jax.jit, static_argnames=())
def kernel(x, event, W1, b1, W2, b2, W3, b3, W4, b4):
    ev3 = event.reshape(NB, 1, P)
    grid = (NB,)
    const = lambda shape: pl.BlockSpec(shape, lambda i: (0,) * len(shape))
    return pl.pallas_call(
        _body,
        grid=grid,
        in_specs=[
            pl.BlockSpec((1, 1, P), lambda i: (i, 0, 0)),
            pl.BlockSpec((P, D), lambda i: (i, 0)),
            const((D, D)),
            const((1, D)),
            const((D, D)),
            const((1, D)),
            const((2 * D, 2 * D)),
            const((1, 2 * D)),
            const((2 * D, G)),
            const((1, G)),
        ],
        out_specs=pl.BlockSpec((NEV, G), lambda i: (0, 0)),
        out_shape=jax.ShapeDtypeStruct((NEV, G), jnp.float32),
        scratch_shapes=[
            pltpu.VMEM((NEV, D), jnp.float32),
            pltpu.VMEM((NEV, 1), jnp.float32),
        ],
        compiler_params=pltpu.CompilerParams(
            dimension_semantics=("arbitrary",),
        ),
    )(ev3, x, W1, b1.reshape(1, D), W2, b2.reshape(1, D),
      W3, b3.reshape(1, 2 * D), W4, b4.reshape(1, G))
